# baseline (speedup 1.0000x reference)
import functools

import jax
import jax.numpy as jnp
from jax import lax
from jax.experimental import pallas as pl
from jax.experimental.pallas import tpu as pltpu
from jax.experimental.pallas import tpu_sc as plsc

D_MODEL = 768
D_FF = 3072
N_EXP = 8
T_TOK = 2048
TILE_M = 128
N_TILES = (T_TOK + N_EXP * TILE_M) // TILE_M
M_PAD = N_TILES * TILE_M
SCALE = 3e-06


def _router_body(x_ref, rw_ref, rb_ref,
                 xs_ref, dest_ref, toff_ref, ntil_ref, loss_ref):
    x = x_ref[...]
    logits = jnp.dot(x, rw_ref[...], preferred_element_type=jnp.float32)
    logits = logits + rb_ref[...]
    lmax = jnp.max(logits, axis=1, keepdims=True)
    p = jnp.exp(logits - lmax)
    probs = p / jnp.sum(p, axis=1, keepdims=True)
    maxp = jnp.max(probs, axis=1, keepdims=True)
    cols = lax.broadcasted_iota(jnp.int32, (T_TOK, N_EXP), 1).astype(jnp.float32)
    eid = jnp.min(jnp.where(probs == maxp, cols, jnp.float32(N_EXP)),
                  axis=1, keepdims=True)
    onehot = (cols == eid).astype(jnp.float32)
    counts = jnp.sum(onehot, axis=0, keepdims=True)

    CH = 256
    r_i = lax.broadcasted_iota(jnp.int32, (CH, CH), 0)
    c_i = lax.broadcasted_iota(jnp.int32, (CH, CH), 1)
    tril = (r_i > c_i).astype(jnp.float32)
    carry = jnp.zeros((1, N_EXP), jnp.float32)
    rank_chunks = []
    for c in range(T_TOK // CH):
        oh = onehot[c * CH:(c + 1) * CH]
        r = jnp.dot(tril, oh, preferred_element_type=jnp.float32) + carry
        rank_chunks.append(jnp.sum(r * oh, axis=1, keepdims=True))
        carry = carry + jnp.sum(oh, axis=0, keepdims=True)
    rank = jnp.concatenate(rank_chunks, axis=0)

    pc = jnp.ceil(counts / TILE_M) * TILE_M
    a_i = lax.broadcasted_iota(jnp.int32, (N_EXP, N_EXP), 0)
    b_i = lax.broadcasted_iota(jnp.int32, (N_EXP, N_EXP), 1)
    excl = (a_i < b_i).astype(jnp.float32)
    offs = jnp.dot(pc, excl, preferred_element_type=jnp.float32)
    dest = jnp.sum(onehot * offs, axis=1, keepdims=True) + rank
    dest_ref[...] = dest.astype(jnp.int32)

    xs_ref[...] = x * maxp

    toff_ref[...] = (offs / TILE_M).reshape(N_EXP, 1).astype(jnp.int32)
    ntil_ref[...] = (pc / TILE_M).reshape(N_EXP, 1).astype(jnp.int32)

    psum = jnp.sum(onehot * maxp, axis=0, keepdims=True)
    loss = jnp.sum((counts / T_TOK) * (psum / (T_TOK * T_TOK)),
                   axis=1, keepdims=True)
    loss_ref[...] = loss * (SCALE * N_EXP)


def _ffn_body(toff_ref, ntil_ref, xs_any, w1_ref, w2_ref, out_any,
              xbuf, ybuf, sem_in, sem_out):
    e = pl.program_id(0)
    base = toff_ref[e, 0] * TILE_M
    n = ntil_ref[e, 0]

    def tile_step(i, carry):
        row0 = base + i * TILE_M
        cp_in = pltpu.make_async_copy(
            xs_any.at[pl.ds(row0, TILE_M)], xbuf, sem_in)
        cp_in.start()
        cp_in.wait()
        h = jnp.dot(xbuf[...], w1_ref[0], preferred_element_type=jnp.float32)
        h = jnp.maximum(h, 0.0)
        ybuf[...] = jnp.dot(h, w2_ref[0], preferred_element_type=jnp.float32)
        cp_out = pltpu.make_async_copy(
            ybuf, out_any.at[pl.ds(row0, TILE_M)], sem_out)
        cp_out.start()
        cp_out.wait()
        return carry

    lax.fori_loop(0, n, tile_step, 0)


def _sc_permute(gather: bool, n_rows_out: int):
    info = plsc.get_sparse_core_info()
    nc, ns = info.num_cores, info.num_subcores
    nw = nc * ns
    bw = T_TOK // nw
    mesh = plsc.VectorSubcoreMesh(core_axis_name="c", subcore_axis_name="s")

    @functools.partial(
        pl.kernel, mesh=mesh,
        out_type=jax.ShapeDtypeStruct((n_rows_out, D_MODEL), jnp.float32),
        scratch_types=[
            pltpu.VMEM((bw,), jnp.int32),
            pltpu.VMEM((bw, D_MODEL), jnp.float32),
            pltpu.SemaphoreType.DMA,
        ],
    )
    def body(rows_hbm, idx_hbm, out_hbm, idx_v, rows_v, sem):
        wid = lax.axis_index("s") * nc + lax.axis_index("c")
        base = wid * bw
        pltpu.sync_copy(idx_hbm.at[pl.ds(base, bw)], idx_v)
        if gather:
            pltpu.async_copy(rows_hbm.at[idx_v], rows_v, sem).wait()
            pltpu.sync_copy(rows_v, out_hbm.at[pl.ds(base, bw)])
        else:
            pltpu.sync_copy(rows_hbm.at[pl.ds(base, bw)], rows_v)
            pltpu.async_copy(rows_v, out_hbm.at[idx_v], sem).wait()

    return body


def kernel(x, w1, w2, router_w, router_b):
    xf = x.reshape(T_TOK, D_MODEL)

    xs, dest, toff, ntil, loss = pl.pallas_call(
        _router_body,
        out_shape=[
            jax.ShapeDtypeStruct((T_TOK, D_MODEL), jnp.float32),
            jax.ShapeDtypeStruct((T_TOK, 1), jnp.int32),
            jax.ShapeDtypeStruct((N_EXP, 1), jnp.int32),
            jax.ShapeDtypeStruct((N_EXP, 1), jnp.int32),
            jax.ShapeDtypeStruct((1, 1), jnp.float32),
        ],
    )(xf, router_w, router_b.reshape(1, N_EXP))

    dest = dest.reshape(T_TOK)
    x_sorted = _sc_permute(gather=False, n_rows_out=M_PAD)(xs, dest)

    grid_spec = pltpu.PrefetchScalarGridSpec(
        num_scalar_prefetch=2,
        grid=(N_EXP,),
        in_specs=[
            pl.BlockSpec(memory_space=pl.ANY),
            pl.BlockSpec((1, D_MODEL, D_FF), lambda e, toff, ntil: (e, 0, 0)),
            pl.BlockSpec((1, D_FF, D_MODEL), lambda e, toff, ntil: (e, 0, 0)),
        ],
        out_specs=pl.BlockSpec(memory_space=pl.ANY),
        scratch_shapes=[
            pltpu.VMEM((TILE_M, D_MODEL), jnp.float32),
            pltpu.VMEM((TILE_M, D_MODEL), jnp.float32),
            pltpu.SemaphoreType.DMA,
            pltpu.SemaphoreType.DMA,
        ],
    )
    out_sorted = pl.pallas_call(
        _ffn_body,
        grid_spec=grid_spec,
        out_shape=jax.ShapeDtypeStruct((M_PAD, D_MODEL), jnp.float32),
    )(toff, ntil, x_sorted, w1, w2)

    out = _sc_permute(gather=True, n_rows_out=T_TOK)(out_sorted, dest)
    return out.reshape(1, T_TOK, D_MODEL), loss.reshape(())


# speedup vs baseline: 1.3813x; 1.3813x over previous
import functools

import jax
import jax.numpy as jnp
from jax import lax
from jax.experimental import pallas as pl
from jax.experimental.pallas import tpu as pltpu
from jax.experimental.pallas import tpu_sc as plsc

D_MODEL = 768
D_FF = 3072
N_EXP = 8
T_TOK = 2048
TILE_M = 128
N_TILES = (T_TOK + N_EXP * TILE_M) // TILE_M
M_PAD = N_TILES * TILE_M
SCALE = 3e-06


def _router_body(x_ref, rw_ref, rb_ref,
                 xs_ref, dest_ref, te_ref, act_ref,
                 fr_ref, sl_ref, nx_ref, ne_ref, loss_ref):
    x = x_ref[...]
    logits = jnp.dot(x, rw_ref[...], preferred_element_type=jnp.float32)
    logits = logits + rb_ref[...]
    lmax = jnp.max(logits, axis=1, keepdims=True)
    p = jnp.exp(logits - lmax)
    probs = p / jnp.sum(p, axis=1, keepdims=True)
    maxp = jnp.max(probs, axis=1, keepdims=True)
    cols = lax.broadcasted_iota(jnp.int32, (T_TOK, N_EXP), 1).astype(jnp.float32)
    eid = jnp.min(jnp.where(probs == maxp, cols, jnp.float32(N_EXP)),
                  axis=1, keepdims=True)
    onehot = (cols == eid).astype(jnp.float32)
    counts = jnp.sum(onehot, axis=0, keepdims=True)

    CH = 256
    r_i = lax.broadcasted_iota(jnp.int32, (CH, CH), 0)
    c_i = lax.broadcasted_iota(jnp.int32, (CH, CH), 1)
    tril = (r_i > c_i).astype(jnp.float32)
    carry = jnp.zeros((1, N_EXP), jnp.float32)
    rank_chunks = []
    for c in range(T_TOK // CH):
        oh = onehot[c * CH:(c + 1) * CH]
        r = jnp.dot(tril, oh, preferred_element_type=jnp.float32) + carry
        rank_chunks.append(jnp.sum(r * oh, axis=1, keepdims=True))
        carry = carry + jnp.sum(oh, axis=0, keepdims=True)
    rank = jnp.concatenate(rank_chunks, axis=0)

    pc = jnp.ceil(counts / TILE_M) * TILE_M
    a_i = lax.broadcasted_iota(jnp.int32, (N_EXP, N_EXP), 0)
    b_i = lax.broadcasted_iota(jnp.int32, (N_EXP, N_EXP), 1)
    excl = (a_i < b_i).astype(jnp.float32)
    offs = jnp.dot(pc, excl, preferred_element_type=jnp.float32)
    dest = jnp.sum(onehot * offs, axis=1, keepdims=True) + rank
    dest_ref[...] = dest.astype(jnp.int32)

    xs_ref[...] = x * maxp

    n_active = jnp.sum(pc) / TILE_M
    k_i = lax.broadcasted_iota(jnp.int32, (N_TILES, 1), 0).astype(jnp.float32)
    kk = jnp.minimum(k_i, n_active - 1.0)
    te = jnp.sum((kk * TILE_M >= offs).astype(jnp.float32), axis=1,
                 keepdims=True) - 1.0
    act = (k_i < n_active).astype(jnp.float32)
    te_ref[...] = te.astype(jnp.int32)
    act_ref[...] = act.astype(jnp.int32)

    cols8 = lax.broadcasted_iota(jnp.int32, (N_TILES, N_EXP), 1).astype(jnp.float32)
    oh_te = (cols8 == te).astype(jnp.float32)
    toff_row = jnp.sum(oh_te * offs, axis=1, keepdims=True)
    fr = act * (k_i * TILE_M == toff_row).astype(jnp.float32)
    nonempty = (counts > 0).astype(jnp.float32)
    incl = (a_i <= b_i).astype(jnp.float32)
    cumne = jnp.dot(nonempty, incl, preferred_element_type=jnp.float32)
    nruns = jnp.sum(nonempty)
    rk = jnp.sum(oh_te * cumne, axis=1, keepdims=True) - 1.0
    sl = rk - 2.0 * jnp.floor(rk * 0.5)
    nr = rk + 1.0
    nx = fr * (nr <= nruns - 1.0).astype(jnp.float32)
    ones8 = jnp.ones((N_EXP, 1), jnp.float32)
    diag8 = (a_i == b_i).astype(jnp.float32)
    ne_col = jnp.dot(diag8 * nonempty, ones8,
                     preferred_element_type=jnp.float32)
    cumne_col = jnp.dot(diag8 * cumne, ones8,
                        preferred_element_type=jnp.float32)
    iota8c = lax.broadcasted_iota(jnp.int32, (N_EXP, N_EXP), 1).astype(jnp.float32)
    ind = (cumne_col - 1.0 == iota8c).astype(jnp.float32) * ne_col
    eids = lax.broadcasted_iota(jnp.int32, (1, N_EXP), 1).astype(jnp.float32)
    re = jnp.dot(eids, ind, preferred_element_type=jnp.float32)
    nr_cl = jnp.minimum(nr, nruns - 1.0)
    ne = jnp.sum((cols8 == nr_cl).astype(jnp.float32) * re,
                 axis=1, keepdims=True)
    fr_ref[...] = fr.astype(jnp.int32)
    sl_ref[...] = sl.astype(jnp.int32)
    nx_ref[...] = nx.astype(jnp.int32)
    ne_ref[...] = ne.astype(jnp.int32)

    psum = jnp.sum(onehot * maxp, axis=0, keepdims=True)
    loss = jnp.sum((counts / T_TOK) * (psum / (T_TOK * T_TOK)),
                   axis=1, keepdims=True)
    loss_ref[...] = loss * (SCALE * N_EXP)


def _ffn_body(te_ref, act_ref, fr_ref, sl_ref, nx_ref, ne_ref,
              xs_ref, w1_any, w2_any, out_ref,
              w1buf, w2buf, sem1, sem2):
    k = pl.program_id(0)
    slot = sl_ref[k, 0]

    @pl.when(k == 0)
    def _():
        e0 = te_ref[0, 0]
        pltpu.make_async_copy(w1_any.at[e0], w1buf.at[0], sem1.at[0]).start()
        pltpu.make_async_copy(w2_any.at[e0], w2buf.at[0], sem2.at[0]).start()

    @pl.when(nx_ref[k, 0] == 1)
    def _():
        nei = ne_ref[k, 0]
        other = 1 - slot
        pltpu.make_async_copy(w1_any.at[nei], w1buf.at[other],
                              sem1.at[other]).start()
        pltpu.make_async_copy(w2_any.at[nei], w2buf.at[other],
                              sem2.at[other]).start()

    @pl.when(fr_ref[k, 0] == 1)
    def _():
        cur = te_ref[k, 0]
        pltpu.make_async_copy(w1_any.at[cur], w1buf.at[slot],
                              sem1.at[slot]).wait()
        pltpu.make_async_copy(w2_any.at[cur], w2buf.at[slot],
                              sem2.at[slot]).wait()

    def compute(s):
        h = jnp.dot(xs_ref[...], w1buf[s], preferred_element_type=jnp.float32)
        h = jnp.maximum(h, 0.0)
        out_ref[...] = jnp.dot(h, w2buf[s], preferred_element_type=jnp.float32)

    @pl.when((act_ref[k, 0] == 1) & (slot == 0))
    def _():
        compute(0)

    @pl.when((act_ref[k, 0] == 1) & (slot == 1))
    def _():
        compute(1)


def _sc_permute(gather: bool, n_rows_out: int):
    info = plsc.get_sparse_core_info()
    nc, ns = info.num_cores, info.num_subcores
    nw = nc * ns
    bw = T_TOK // nw
    mesh = plsc.VectorSubcoreMesh(core_axis_name="c", subcore_axis_name="s")

    @functools.partial(
        pl.kernel, mesh=mesh,
        out_type=jax.ShapeDtypeStruct((n_rows_out, D_MODEL), jnp.float32),
        scratch_types=[
            pltpu.VMEM((bw,), jnp.int32),
            pltpu.VMEM((bw, D_MODEL), jnp.float32),
            pltpu.SemaphoreType.DMA,
        ],
    )
    def body(rows_hbm, idx_hbm, out_hbm, idx_v, rows_v, sem):
        wid = lax.axis_index("s") * nc + lax.axis_index("c")
        base = wid * bw
        pltpu.sync_copy(idx_hbm.at[pl.ds(base, bw)], idx_v)
        if gather:
            pltpu.async_copy(rows_hbm.at[idx_v], rows_v, sem).wait()
            pltpu.sync_copy(rows_v, out_hbm.at[pl.ds(base, bw)])
        else:
            pltpu.sync_copy(rows_hbm.at[pl.ds(base, bw)], rows_v)
            pltpu.async_copy(rows_v, out_hbm.at[idx_v], sem).wait()

    return body


def kernel(x, w1, w2, router_w, router_b):
    xf = x.reshape(T_TOK, D_MODEL)

    xs, dest, te, act, fr, sl, nx, ne, loss = pl.pallas_call(
        _router_body,
        out_shape=[
            jax.ShapeDtypeStruct((T_TOK, D_MODEL), jnp.float32),
            jax.ShapeDtypeStruct((T_TOK, 1), jnp.int32),
            jax.ShapeDtypeStruct((N_TILES, 1), jnp.int32),
            jax.ShapeDtypeStruct((N_TILES, 1), jnp.int32),
            jax.ShapeDtypeStruct((N_TILES, 1), jnp.int32),
            jax.ShapeDtypeStruct((N_TILES, 1), jnp.int32),
            jax.ShapeDtypeStruct((N_TILES, 1), jnp.int32),
            jax.ShapeDtypeStruct((N_TILES, 1), jnp.int32),
            jax.ShapeDtypeStruct((1, 1), jnp.float32),
        ],
    )(xf, router_w, router_b.reshape(1, N_EXP))

    dest = dest.reshape(T_TOK)
    x_sorted = _sc_permute(gather=False, n_rows_out=M_PAD)(xs, dest)

    grid_spec = pltpu.PrefetchScalarGridSpec(
        num_scalar_prefetch=6,
        grid=(N_TILES,),
        in_specs=[
            pl.BlockSpec((TILE_M, D_MODEL),
                         lambda k, *refs: (k, 0)),
            pl.BlockSpec(memory_space=pl.ANY),
            pl.BlockSpec(memory_space=pl.ANY),
        ],
        out_specs=pl.BlockSpec((TILE_M, D_MODEL), lambda k, *refs: (k, 0)),
        scratch_shapes=[
            pltpu.VMEM((2, D_MODEL, D_FF), jnp.float32),
            pltpu.VMEM((2, D_FF, D_MODEL), jnp.float32),
            pltpu.SemaphoreType.DMA((2,)),
            pltpu.SemaphoreType.DMA((2,)),
        ],
    )
    out_sorted = pl.pallas_call(
        _ffn_body,
        grid_spec=grid_spec,
        out_shape=jax.ShapeDtypeStruct((M_PAD, D_MODEL), jnp.float32),
    )(te, act, fr, sl, nx, ne, x_sorted, w1, w2)

    out = _sc_permute(gather=True, n_rows_out=T_TOK)(out_sorted, dest)
    return out.reshape(1, T_TOK, D_MODEL), loss.reshape(())


# speedup vs baseline: 1.4193x; 1.0275x over previous
import functools

import jax
import jax.numpy as jnp
from jax import lax
from jax.experimental import pallas as pl
from jax.experimental.pallas import tpu as pltpu
from jax.experimental.pallas import tpu_sc as plsc

D_MODEL = 768
D_FF = 3072
N_EXP = 8
T_TOK = 2048
TILE_M = 128
N_TILES = (T_TOK + N_EXP * TILE_M) // TILE_M
M_PAD = N_TILES * TILE_M
SCALE = 3e-06


def _router_body(x_ref, rw_ref, rb_ref,
                 xs_ref, dest_ref, te_ref, act_ref,
                 fr_ref, sl_ref, nx_ref, ne_ref, loss_ref):
    x = x_ref[...]
    logits = jnp.dot(x, rw_ref[...], preferred_element_type=jnp.float32)
    logits = logits + rb_ref[...]
    lmax = jnp.max(logits, axis=1, keepdims=True)
    p = jnp.exp(logits - lmax)
    probs = p / jnp.sum(p, axis=1, keepdims=True)
    maxp = jnp.max(probs, axis=1, keepdims=True)
    cols = lax.broadcasted_iota(jnp.int32, (T_TOK, N_EXP), 1).astype(jnp.float32)
    eid = jnp.min(jnp.where(probs == maxp, cols, jnp.float32(N_EXP)),
                  axis=1, keepdims=True)
    onehot = (cols == eid).astype(jnp.float32)
    counts = jnp.sum(onehot, axis=0, keepdims=True)

    CH = 256
    r_i = lax.broadcasted_iota(jnp.int32, (CH, CH), 0)
    c_i = lax.broadcasted_iota(jnp.int32, (CH, CH), 1)
    tril = (r_i > c_i).astype(jnp.float32)
    carry = jnp.zeros((1, N_EXP), jnp.float32)
    rank_chunks = []
    for c in range(T_TOK // CH):
        oh = onehot[c * CH:(c + 1) * CH]
        r = jnp.dot(tril, oh, preferred_element_type=jnp.float32) + carry
        rank_chunks.append(jnp.sum(r * oh, axis=1, keepdims=True))
        carry = carry + jnp.sum(oh, axis=0, keepdims=True)
    rank = jnp.concatenate(rank_chunks, axis=0)

    pc = jnp.ceil(counts / TILE_M) * TILE_M
    a_i = lax.broadcasted_iota(jnp.int32, (N_EXP, N_EXP), 0)
    b_i = lax.broadcasted_iota(jnp.int32, (N_EXP, N_EXP), 1)
    excl = (a_i < b_i).astype(jnp.float32)
    offs = jnp.dot(pc, excl, preferred_element_type=jnp.float32)
    dest = jnp.sum(onehot * offs, axis=1, keepdims=True) + rank
    dest_ref[...] = dest.astype(jnp.int32).reshape(1, T_TOK)

    xs_ref[...] = x * maxp

    n_active = jnp.sum(pc) / TILE_M
    k_i = lax.broadcasted_iota(jnp.int32, (N_TILES, 1), 0).astype(jnp.float32)
    kk = jnp.minimum(k_i, n_active - 1.0)
    te = jnp.sum((kk * TILE_M >= offs).astype(jnp.float32), axis=1,
                 keepdims=True) - 1.0
    act = (k_i < n_active).astype(jnp.float32)
    te_ref[...] = te.astype(jnp.int32)
    act_ref[...] = act.astype(jnp.int32)

    cols8 = lax.broadcasted_iota(jnp.int32, (N_TILES, N_EXP), 1).astype(jnp.float32)
    oh_te = (cols8 == te).astype(jnp.float32)
    toff_row = jnp.sum(oh_te * offs, axis=1, keepdims=True)
    fr = act * (k_i * TILE_M == toff_row).astype(jnp.float32)
    nonempty = (counts > 0).astype(jnp.float32)
    incl = (a_i <= b_i).astype(jnp.float32)
    cumne = jnp.dot(nonempty, incl, preferred_element_type=jnp.float32)
    nruns = jnp.sum(nonempty)
    rk = jnp.sum(oh_te * cumne, axis=1, keepdims=True) - 1.0
    sl = rk - 2.0 * jnp.floor(rk * 0.5)
    nr = rk + 1.0
    nx = fr * (nr <= nruns - 1.0).astype(jnp.float32)
    ones8 = jnp.ones((N_EXP, 1), jnp.float32)
    diag8 = (a_i == b_i).astype(jnp.float32)
    ne_col = jnp.dot(diag8 * nonempty, ones8,
                     preferred_element_type=jnp.float32)
    cumne_col = jnp.dot(diag8 * cumne, ones8,
                        preferred_element_type=jnp.float32)
    iota8c = lax.broadcasted_iota(jnp.int32, (N_EXP, N_EXP), 1).astype(jnp.float32)
    ind = (cumne_col - 1.0 == iota8c).astype(jnp.float32) * ne_col
    eids = lax.broadcasted_iota(jnp.int32, (1, N_EXP), 1).astype(jnp.float32)
    re = jnp.dot(eids, ind, preferred_element_type=jnp.float32)
    nr_cl = jnp.minimum(nr, nruns - 1.0)
    ne = jnp.sum((cols8 == nr_cl).astype(jnp.float32) * re,
                 axis=1, keepdims=True)
    fr_ref[...] = fr.astype(jnp.int32)
    sl_ref[...] = sl.astype(jnp.int32)
    nx_ref[...] = nx.astype(jnp.int32)
    ne_ref[...] = ne.astype(jnp.int32)

    psum = jnp.sum(onehot * maxp, axis=0, keepdims=True)
    loss = jnp.sum((counts / T_TOK) * (psum / (T_TOK * T_TOK)),
                   axis=1, keepdims=True)
    loss_ref[...] = loss * (SCALE * N_EXP)


def _ffn_body(te_ref, act_ref, fr_ref, sl_ref, nx_ref, ne_ref,
              xs_ref, w1_any, w2_any, out_ref,
              w1buf, w2buf, sem1, sem2):
    k = pl.program_id(0)
    slot = sl_ref[k, 0]

    @pl.when(k == 0)
    def _():
        e0 = te_ref[0, 0]
        pltpu.make_async_copy(w1_any.at[e0], w1buf.at[0], sem1.at[0]).start()
        pltpu.make_async_copy(w2_any.at[e0], w2buf.at[0], sem2.at[0]).start()

    @pl.when(nx_ref[k, 0] == 1)
    def _():
        nei = ne_ref[k, 0]
        other = 1 - slot
        pltpu.make_async_copy(w1_any.at[nei], w1buf.at[other],
                              sem1.at[other]).start()
        pltpu.make_async_copy(w2_any.at[nei], w2buf.at[other],
                              sem2.at[other]).start()

    @pl.when(fr_ref[k, 0] == 1)
    def _():
        cur = te_ref[k, 0]
        pltpu.make_async_copy(w1_any.at[cur], w1buf.at[slot],
                              sem1.at[slot]).wait()
        pltpu.make_async_copy(w2_any.at[cur], w2buf.at[slot],
                              sem2.at[slot]).wait()

    def compute(s):
        h = jnp.dot(xs_ref[...], w1buf[s], preferred_element_type=jnp.float32)
        h = jnp.maximum(h, 0.0)
        out_ref[...] = jnp.dot(h, w2buf[s], preferred_element_type=jnp.float32)

    @pl.when((act_ref[k, 0] == 1) & (slot == 0))
    def _():
        compute(0)

    @pl.when((act_ref[k, 0] == 1) & (slot == 1))
    def _():
        compute(1)


def _sc_permute(gather: bool, n_rows_out: int, dtype=jnp.float32):
    info = plsc.get_sparse_core_info()
    nc, ns = info.num_cores, info.num_subcores
    nw = nc * ns
    bw = T_TOK // nw
    mesh = plsc.VectorSubcoreMesh(core_axis_name="c", subcore_axis_name="s")

    @functools.partial(
        pl.kernel, mesh=mesh,
        out_type=jax.ShapeDtypeStruct((n_rows_out, D_MODEL), dtype),
        scratch_types=[
            pltpu.VMEM((bw,), jnp.int32),
            pltpu.VMEM((bw, D_MODEL), dtype),
            pltpu.SemaphoreType.DMA,
        ],
    )
    def body(rows_hbm, idx_hbm, out_hbm, idx_v, rows_v, sem):
        wid = lax.axis_index("s") * nc + lax.axis_index("c")
        base = wid * bw
        pltpu.sync_copy(idx_hbm.at[pl.ds(base, bw)], idx_v)
        if gather:
            pltpu.async_copy(rows_hbm.at[idx_v], rows_v, sem).wait()
            pltpu.sync_copy(rows_v, out_hbm.at[pl.ds(base, bw)])
        else:
            pltpu.sync_copy(rows_hbm.at[pl.ds(base, bw)], rows_v)
            pltpu.async_copy(rows_v, out_hbm.at[idx_v], sem).wait()

    return body


def kernel(x, w1, w2, router_w, router_b):
    xf = x.reshape(T_TOK, D_MODEL)

    xs, dest, te, act, fr, sl, nx, ne, loss = pl.pallas_call(
        _router_body,
        out_shape=[
            jax.ShapeDtypeStruct((T_TOK, D_MODEL), jnp.float32),
            jax.ShapeDtypeStruct((1, T_TOK), jnp.int32),
            jax.ShapeDtypeStruct((N_TILES, 1), jnp.int32),
            jax.ShapeDtypeStruct((N_TILES, 1), jnp.int32),
            jax.ShapeDtypeStruct((N_TILES, 1), jnp.int32),
            jax.ShapeDtypeStruct((N_TILES, 1), jnp.int32),
            jax.ShapeDtypeStruct((N_TILES, 1), jnp.int32),
            jax.ShapeDtypeStruct((N_TILES, 1), jnp.int32),
            jax.ShapeDtypeStruct((1, 1), jnp.float32),
        ],
    )(xf, router_w, router_b.reshape(1, N_EXP))

    dest = dest.reshape(T_TOK)
    x_sorted = _sc_permute(gather=False, n_rows_out=M_PAD)(xs, dest)

    grid_spec = pltpu.PrefetchScalarGridSpec(
        num_scalar_prefetch=6,
        grid=(N_TILES,),
        in_specs=[
            pl.BlockSpec((TILE_M, D_MODEL),
                         lambda k, *refs: (k, 0)),
            pl.BlockSpec(memory_space=pl.ANY),
            pl.BlockSpec(memory_space=pl.ANY),
        ],
        out_specs=pl.BlockSpec((TILE_M, D_MODEL), lambda k, *refs: (k, 0)),
        scratch_shapes=[
            pltpu.VMEM((2, D_MODEL, D_FF), jnp.float32),
            pltpu.VMEM((2, D_FF, D_MODEL), jnp.float32),
            pltpu.SemaphoreType.DMA((2,)),
            pltpu.SemaphoreType.DMA((2,)),
        ],
    )
    out_sorted = pl.pallas_call(
        _ffn_body,
        grid_spec=grid_spec,
        out_shape=jax.ShapeDtypeStruct((M_PAD, D_MODEL), jnp.float32),
    )(te, act, fr, sl, nx, ne, x_sorted, w1, w2)

    out = _sc_permute(gather=True, n_rows_out=T_TOK)(out_sorted, dest)
    return out.reshape(1, T_TOK, D_MODEL), loss.reshape(())
